# Initial kernel scaffold; baseline (speedup 1.0000x reference)
import jax
import jax.numpy as jnp
from jax.experimental import pallas as pl


def kernel(points, outputs):
    raise NotImplementedError("write your pallas kernel here")


# speedup vs baseline: 56.6688x; 56.6688x over previous
import functools

import jax
import jax.numpy as jnp
from jax.experimental import pallas as pl

_N = 1000
_NP = 1024
_NC = 10
_NCP = 128
_K = _N + (_N * _N - _N - 1) // 2
_LAMBDA_SPEC = 0.05
_EPS = 1e-8


def _spectral_loss_kernel(pxc, pyc, pzc, pxr, pyr, pzr, u_ref, out_ref):
    dx = pxc[:] - pxr[:]
    dy = pyc[:] - pyr[:]
    dz = pzc[:] - pzr[:]
    sq = dx * dx + dy * dy + dz * dz

    rows = jax.lax.broadcasted_iota(jnp.int32, (_NP, _NP), 0)
    cols = jax.lax.broadcasted_iota(jnp.int32, (_NP, _NP), 1)
    valid = (rows < _N) & (cols < _N)

    bits = jax.lax.bitcast_convert_type(sq, jnp.int32)
    bits = jnp.where(valid, bits, jnp.int32(0x7FFFFFFF))

    def body(i, res):
        trial = res | (jnp.int32(1) << (jnp.int32(30) - i))
        cnt = jnp.sum((bits < trial).astype(jnp.int32))
        return jnp.where(cnt <= _K, trial, res)

    med_bits = jax.lax.fori_loop(0, 31, body, jnp.int32(0))
    sigma_sq = jax.lax.bitcast_convert_type(med_bits, jnp.float32)
    sigma = jnp.sqrt(sigma_sq)
    denom = 2.0 * sigma * sigma + _EPS

    mask_w = valid & (rows != cols)
    w = jnp.where(mask_w, jnp.exp(-sq / denom), 0.0)

    d = jnp.sum(w, axis=1, keepdims=True)
    dinv = 1.0 / (jnp.sqrt(d) + _EPS)
    u = u_ref[:] * dinv
    v = jnp.dot(w, u, preferred_element_type=jnp.float32)
    s2 = jnp.sum(v * u)
    s1 = jnp.sum(u_ref[:] * u_ref[:])
    total = (s1 - s2) / _NC
    loss = _LAMBDA_SPEC * total / (_N * _N)
    out_ref[:, :] = jnp.full((1, 1), loss, dtype=jnp.float32)


@jax.jit
def kernel(points, outputs):
    pc = jnp.pad(points, ((0, _NP - _N), (0, 0)))
    pt = pc.T
    u = jnp.pad(outputs, ((0, _NP - _N), (0, _NCP - _NC)))
    args = [
        pc[:, 0:1], pc[:, 1:2], pc[:, 2:3],
        pt[0:1, :], pt[1:2, :], pt[2:3, :],
        u,
    ]
    out = pl.pallas_call(
        _spectral_loss_kernel,
        out_shape=jax.ShapeDtypeStruct((1, 1), jnp.float32),
    )(*args)
    return out[0, 0]
